# Initial kernel scaffold; baseline (speedup 1.0000x reference)
import jax
import jax.numpy as jnp
from jax.experimental import pallas as pl


def kernel(x_list, batch_list, edge_index, enc0_W1, enc0_b1, enc0_g, enc0_be, enc0_W2, enc0_b2, dec0_W, dec0_b, enc1_W1, enc1_b1, enc1_g, enc1_be, enc1_W2, enc1_b2, dec1_W, dec1_b, c_W1, c_b1, c_g, c_be, c_W2, c_b2, clf_W):
    raise NotImplementedError("write your pallas kernel here")


# speedup vs baseline: 7.3386x; 7.3386x over previous
import functools

import jax
import jax.numpy as jnp
from jax import lax
from jax.experimental import pallas as pl
from jax.experimental.pallas import tpu as pltpu
from jax.experimental.pallas import tpu_sc as plsc

_N = 10000
_E = 160000
_NC = 2
_NS = 16
_EPT = _E // _NS
_CH = 80
_NCHUNK = _EPT // _CH
_RPT = _N // _NS
_RCH = 25
_NRCH = _RPT // _RCH

_R = 1000
_NB = _N // _R

_mesh = lambda: plsc.VectorSubcoreMesh(core_axis_name="c", subcore_axis_name="s")
_sc_params = lambda: pltpu.CompilerParams(use_tc_tiling_on_sc=False)


@functools.lru_cache(maxsize=None)
def _make_sc_prop(fh):
  @functools.partial(
      pl.kernel,
      out_type=jax.ShapeDtypeStruct((_NC, _N, fh), jnp.float32),
      mesh=_mesh(),
      scratch_types=[
          pltpu.VMEM((_CH,), jnp.int32),
          pltpu.VMEM((_CH,), jnp.int32),
          pltpu.VMEM((_CH, fh), jnp.float32),
          pltpu.VMEM((_RCH, fh), jnp.float32),
          pltpu.VMEM_SHARED((_N, fh), jnp.float32),
          pltpu.SemaphoreType.DMA,
      ],
      compiler_params=_sc_params(),
  )
  def prop(src2_hbm, dst_hbm, z_hbm, zeros_hbm, out_hbm,
           gidx_v, didx_v, rows_v, buf_v, acc_sp, sem):
    c = lax.axis_index("c")
    s = lax.axis_index("s")

    pltpu.sync_copy(zeros_hbm, buf_v)
    for q in range(_NRCH):
      pltpu.sync_copy(buf_v, acc_sp.at[pl.ds(s * _RPT + q * _RCH, _RCH)])
    plsc.subcore_barrier()

    ebase = s * _EPT

    @pl.loop(0, _NCHUNK)
    def _chunk(k):
      base = ebase + k * _CH
      pltpu.sync_copy(src2_hbm.at[c, pl.ds(base, _CH)], gidx_v)
      pltpu.sync_copy(dst_hbm.at[pl.ds(base, _CH)], didx_v)
      pltpu.async_copy(z_hbm.at[gidx_v], rows_v, sem).wait()
      pltpu.sync_copy(rows_v, acc_sp.at[didx_v], add=True)

    plsc.subcore_barrier()
    for q in range(_NRCH):
      r0 = s * _RPT + q * _RCH
      pltpu.sync_copy(acc_sp.at[pl.ds(r0, _RCH)], buf_v)
      pltpu.sync_copy(buf_v, out_hbm.at[c, pl.ds(r0, _RCH)])

  return prop


@functools.lru_cache(maxsize=None)
def _make_sc_deg():
  @functools.partial(
      pl.kernel,
      out_type=jax.ShapeDtypeStruct((_NC, _N, 16), jnp.float32),
      mesh=_mesh(),
      scratch_types=[
          pltpu.VMEM((_CH,), jnp.int32),
          pltpu.VMEM((_CH, 16), jnp.float32),
          pltpu.VMEM((_RCH, 16), jnp.float32),
          pltpu.VMEM_SHARED((_N, 16), jnp.float32),
      ],
      compiler_params=_sc_params(),
  )
  def _sc_deg(dst_hbm, ones_hbm, out_hbm, didx_v, pay_v, buf_v, acc_sp):
    c = lax.axis_index("c")
    s = lax.axis_index("s")

    pltpu.sync_copy(ones_hbm.at[pl.ds(0, _RCH)], buf_v)
    for q in range(_NRCH):
      pltpu.sync_copy(buf_v, acc_sp.at[pl.ds(s * _RPT + q * _RCH, _RCH)])
    pltpu.sync_copy(ones_hbm, pay_v)
    plsc.subcore_barrier()

    ebase = s * _EPT

    @pl.loop(0, _NCHUNK)
    def _chunk(k):
      base = ebase + k * _CH
      pltpu.sync_copy(dst_hbm.at[pl.ds(base, _CH)], didx_v)
      pltpu.sync_copy(pay_v, acc_sp.at[didx_v], add=True)

    plsc.subcore_barrier()
    for q in range(_NRCH):
      r0 = s * _RPT + q * _RCH
      pltpu.sync_copy(acc_sp.at[pl.ds(r0, _RCH)], buf_v)
      pltpu.sync_copy(buf_v, out_hbm.at[c, pl.ds(r0, _RCH)])

  return _sc_deg


def _tc1_body(x_ref, deg_ref, w1_ref, z1_ref):
  dinv = lax.rsqrt(deg_ref[0, :, 0:1])
  for c in range(2):
    z1_ref[c] = jnp.dot(x_ref[c], w1_ref[c],
                        preferred_element_type=jnp.float32) * dinv


def _tc2_body(s1_ref, z1_ref, deg_ref, b1_ref, g_ref, be_ref, w2_ref, z2_ref):
  dinv = lax.rsqrt(deg_ref[0, :, 0:1])
  for c in range(2):
    t = dinv * (s1_ref[c] + z1_ref[c]) + b1_ref[c]
    mu = jnp.mean(t, axis=-1, keepdims=True)
    var = jnp.mean((t - mu) ** 2, axis=-1, keepdims=True)
    h = jnp.tanh((t - mu) / jnp.sqrt(var + 1e-5) * g_ref[c] + be_ref[c])
    z2_ref[c] = jnp.dot(h, w2_ref[c],
                        preferred_element_type=jnp.float32) * dinv


def _tc3_body(s2_ref, z2_ref, deg_ref, b2_ref, clf_ref, dw_ref, cw1_ref,
              comb_ref, dom_ref, z3_ref):
  dinv = lax.rsqrt(deg_ref[0, :, 0:1])
  feats = [dinv * (s2_ref[c] + z2_ref[c]) + b2_ref[c] for c in range(2)]
  comb = jnp.concatenate(feats, axis=1)
  comb_ref[...] = comb
  clf = clf_ref[...]
  wn = clf / jnp.maximum(
      jnp.sqrt(jnp.sum(clf * clf, axis=0, keepdims=True)), 1e-12)
  cc = jnp.dot(comb, cw1_ref[...], preferred_element_type=jnp.float32)
  for c in range(2):
    f = feats[c]
    fn = f / jnp.maximum(
        jnp.sqrt(jnp.sum(f * f, axis=1, keepdims=True)), 1e-12)
    dom_ref[c] = 5.0 * jnp.dot(fn, wn, preferred_element_type=jnp.float32)
    dec = jnp.dot(f, dw_ref[c], preferred_element_type=jnp.float32)
    z3_ref[c] = jnp.concatenate(
        [dec, cc[:, c * 32:(c + 1) * 32]], axis=1) * dinv


def _tc4_body(s3_ref, z3_ref, deg_ref, db_ref, cb1_ref, cg_ref, cbe_ref,
              cw2_ref, rec_ref, z4_ref):
  dinv = lax.rsqrt(deg_ref[0, :, 0:1])
  for c in range(2):
    rec_ref[c] = dinv * (s3_ref[c][:, :128] + z3_ref[c][:, :128]) + db_ref[c]
  t = jnp.concatenate([s3_ref[0][:, 128:] + z3_ref[0][:, 128:],
                       s3_ref[1][:, 128:] + z3_ref[1][:, 128:]], axis=1)
  t = dinv * t + cb1_ref[0]
  mask = (lax.broadcasted_iota(jnp.int32, t.shape, 1) < 42).astype(jnp.float32)
  mu = jnp.sum(t, axis=-1, keepdims=True) / 42.0
  d = (t - mu) * mask
  var = jnp.sum(d * d, axis=-1, keepdims=True) / 42.0
  h = jnp.maximum(d / jnp.sqrt(var + 1e-5) * cg_ref[0] + cbe_ref[0], 0.0)
  for c in range(2):
    z4_ref[c] = jnp.dot(h, cw2_ref[c],
                        preferred_element_type=jnp.float32) * dinv


def _tc5_body(s4_ref, z4_ref, deg_ref, cb2_ref, out_ref):
  dinv = lax.rsqrt(deg_ref[0, :, 0:1])
  out_ref[...] = dinv * jnp.concatenate(
      [s4_ref[0] + z4_ref[0], s4_ref[1] + z4_ref[1]], axis=1) + cb2_ref[0]


def _rows(shape):
  return pl.BlockSpec((shape[0], _R) + shape[2:], lambda i: (0, i) + (0,) * (len(shape) - 2))


def _rows2(shape):
  return pl.BlockSpec((_R,) + shape[1:], lambda i: (i,) + (0,) * (len(shape) - 1))


def _full(shape):
  return pl.BlockSpec(shape, lambda i: (0,) * len(shape))


def _tc_call(body, in_arrays, out_shapes, row_split):
  arrs = list(in_arrays)
  specs = []
  for a, split in zip(arrs + list(out_shapes), row_split):
    shape = a.shape
    if split:
      specs.append(_rows(shape) if len(shape) == 3 else _rows2(shape))
    else:
      specs.append(_full(shape))
  n_in = len(arrs)
  return pl.pallas_call(
      body,
      grid=(_NB,),
      in_specs=specs[:n_in],
      out_specs=specs[n_in] if len(out_shapes) == 1 else specs[n_in:],
      out_shape=(out_shapes[0] if len(out_shapes) == 1
                 else tuple(out_shapes)),
  )(*arrs)


def kernel(x_list, batch_list, edge_index, enc0_W1, enc0_b1, enc0_g, enc0_be,
           enc0_W2, enc0_b2, dec0_W, dec0_b, enc1_W1, enc1_b1, enc1_g,
           enc1_be, enc1_W2, enc1_b2, dec1_W, dec1_b, c_W1, c_b1, c_g, c_be,
           c_W2, c_b2, clf_W):
  f32 = jnp.float32
  src = edge_index[0]
  dst = edge_index[1]
  src2 = jnp.stack([src, src + _N])
  ones16 = jnp.ones((_CH, 16), f32)

  w1s = jnp.stack([enc0_W1, enc1_W1])
  b1s = jnp.stack([enc0_b1, enc1_b1])
  gs = jnp.stack([enc0_g, enc1_g])
  bes = jnp.stack([enc0_be, enc1_be])
  w2s = jnp.stack([enc0_W2, enc1_W2])
  b2s = jnp.stack([enc0_b2, enc1_b2])
  dws = jnp.stack([dec0_W, dec1_W])
  dbs = jnp.stack([dec0_b, dec1_b])
  cw1p = jnp.pad(c_W1, ((0, 0), (0, 22)))
  cb1p = jnp.pad(c_b1, (0, 22)).reshape(1, 64)
  cgp = jnp.pad(c_g, (0, 22)).reshape(1, 64)
  cbep = jnp.pad(c_be, (0, 22)).reshape(1, 64)
  cw2p = jnp.pad(c_W2, ((0, 22), (0, 0)))
  cw2s = jnp.stack([cw2p[:, :32], cw2p[:, 32:]])
  cb2r = c_b2.reshape(1, 64)

  deg = _make_sc_deg()(dst, ones16)

  sds = lambda *s: jax.ShapeDtypeStruct(s, f32)

  z1 = _tc_call(_tc1_body, [x_list, deg, w1s], [sds(2, _N, 128)],
                [1, 1, 0, 1])
  s1 = _make_sc_prop(128)(src2, dst, z1.reshape(2 * _N, 128),
                   jnp.zeros((_RCH, 128), f32))
  z2 = _tc_call(_tc2_body, [s1, z1, deg, b1s, gs, bes, w2s],
                [sds(2, _N, 64)], [1, 1, 1, 0, 0, 0, 0, 1])
  s2 = _make_sc_prop(64)(src2, dst, z2.reshape(2 * _N, 64),
                         jnp.zeros((_RCH, 64), f32))
  comb, doms, z3 = _tc_call(
      _tc3_body, [s2, z2, deg, b2s, clf_W, dws, cw1p],
      [sds(_N, 128), sds(2, _N, 8), sds(2, _N, 160)],
      [1, 1, 1, 0, 0, 0, 0, 1, 1, 1])
  s3 = _make_sc_prop(160)(src2, dst, z3.reshape(2 * _N, 160),
                          jnp.zeros((_RCH, 160), f32))
  recs, z4 = _tc_call(
      _tc4_body, [s3, z3, deg, dbs, cb1p, cgp, cbep, cw2s],
      [sds(2, _N, 128), sds(2, _N, 32)],
      [1, 1, 1, 0, 0, 0, 0, 0, 1, 1])
  s4 = _make_sc_prop(32)(src2, dst, z4.reshape(2 * _N, 32),
                         jnp.zeros((_RCH, 32), f32))
  comb_out = _tc_call(_tc5_body, [s4, z4, deg, cb2r], [sds(_N, 64)],
                      [1, 1, 1, 0, 1])

  return (comb[:, :64], comb[:, 64:], doms[0], doms[1],
          recs[0], recs[1], comb_out)


# speedup vs baseline: 15.3689x; 2.0942x over previous
import functools

import jax
import jax.numpy as jnp
from jax import lax
from jax.experimental import pallas as pl
from jax.experimental.pallas import tpu as pltpu
from jax.experimental.pallas import tpu_sc as plsc

_N = 10000
_E = 160000
_NC = 2
_NS = 16
_CH = 40
_NCROWS = _E // _CH
_CPT = _NCROWS // _NS
_BLK = 50
_NBLK = _CPT // _BLK
_RPT = _N // _NS

_R = 1000
_NB = _N // _R

_mesh = lambda: plsc.VectorSubcoreMesh(core_axis_name="c", subcore_axis_name="s")
_sc_params = lambda: pltpu.CompilerParams(use_tc_tiling_on_sc=False)


@functools.lru_cache(maxsize=None)
def _make_sc_prop(fh):
  nbuf = 2 if fh > 128 else 5

  @functools.partial(
      pl.kernel,
      out_type=jax.ShapeDtypeStruct((_NC, _N, fh), jnp.float32),
      mesh=_mesh(),
      scratch_types=[
          pltpu.VMEM((_BLK, _CH), jnp.int32),
          pltpu.VMEM((_BLK, _CH), jnp.int32),
          pltpu.VMEM((nbuf, _CH, fh), jnp.float32),
          pltpu.VMEM_SHARED((_N, fh), jnp.float32),
          pltpu.SemaphoreType.DMA,
      ] + [pltpu.SemaphoreType.DMA] * (2 * nbuf),
      compiler_params=_sc_params(),
  )
  def prop(src2_hbm, dst_hbm, z_hbm, zeros_hbm, out_hbm,
           srci_v, dsti_v, rows_v, acc_sp, isem, *sems):
    gsem = sems[:nbuf]
    ssem = sems[nbuf:]
    c = lax.axis_index("c")
    s = lax.axis_index("s")

    pltpu.async_copy(zeros_hbm, acc_sp.at[pl.ds(s * _RPT, _RPT)], isem).wait()
    plsc.subcore_barrier()

    def gath(krow, b):
      pltpu.async_copy(z_hbm.at[srci_v.at[krow]], rows_v.at[b], gsem[b])

    def wait_gath(b):
      pltpu.make_async_copy(
          z_hbm.at[srci_v.at[0]], rows_v.at[b], gsem[b]).wait()

    def scat(jrow, b):
      pltpu.async_copy(rows_v.at[b], acc_sp.at[dsti_v.at[jrow]], ssem[b],
                       add=True)

    def wait_scat(b):
      pltpu.make_async_copy(
          rows_v.at[b], acc_sp.at[dsti_v.at[0]], ssem[b]).wait()

    for blk in range(_NBLK):
      r0 = s * _CPT + blk * _BLK
      pltpu.sync_copy(src2_hbm.at[c, pl.ds(r0, _BLK)], srci_v)
      pltpu.sync_copy(dst_hbm.at[pl.ds(r0, _BLK)], dsti_v)

      for k in range(nbuf):
        gath(k, k)
      wait_gath(0)
      scat(0, 0)

      @pl.loop(0, (_BLK - nbuf) // nbuf)
      def _grp(g):
        for i in range(nbuf):
          k = nbuf + g * nbuf + i
          wait_scat(i)
          gath(k, i)
          bs = (i + 1) % nbuf
          wait_gath(bs)
          scat(k - nbuf + 1, bs)

      for j in range(_BLK - nbuf + 1, _BLK):
        b = j % nbuf
        wait_gath(b)
        scat(j, b)
      for b in range(nbuf):
        wait_scat(b)

    plsc.subcore_barrier()
    pltpu.async_copy(acc_sp.at[pl.ds(s * _RPT, _RPT)],
                     out_hbm.at[c, pl.ds(s * _RPT, _RPT)], isem).wait()

  return prop


@functools.lru_cache(maxsize=None)
def _make_sc_deg():
  @functools.partial(
      pl.kernel,
      out_type=jax.ShapeDtypeStruct((_NC, _N, 16), jnp.float32),
      mesh=_mesh(),
      scratch_types=[
          pltpu.VMEM((_BLK, _CH), jnp.int32),
          pltpu.VMEM((_CH, 16), jnp.float32),
          pltpu.VMEM_SHARED((_N, 16), jnp.float32),
          pltpu.SemaphoreType.DMA,
          pltpu.SemaphoreType.DMA,
      ],
      compiler_params=_sc_params(),
  )
  def _sc_deg(dst_hbm, ones_hbm, out_hbm, didx_v, pay_v, acc_sp, isem, ssem):
    c = lax.axis_index("c")
    s = lax.axis_index("s")

    pltpu.async_copy(ones_hbm, acc_sp.at[pl.ds(s * _RPT, _RPT)], isem).wait()
    pltpu.sync_copy(ones_hbm.at[pl.ds(0, _CH)], pay_v)
    plsc.subcore_barrier()

    for blk in range(_NBLK):
      r0 = s * _CPT + blk * _BLK
      pltpu.sync_copy(dst_hbm.at[pl.ds(r0, _BLK)], didx_v)

      @pl.loop(0, _BLK // 10)
      def _grp(g):
        for i in range(10):
          pltpu.async_copy(pay_v, acc_sp.at[didx_v.at[g * 10 + i]], ssem,
                           add=True)
        for i in range(10):
          pltpu.make_async_copy(pay_v, acc_sp.at[didx_v.at[0]], ssem).wait()

    plsc.subcore_barrier()
    pltpu.async_copy(acc_sp.at[pl.ds(s * _RPT, _RPT)],
                     out_hbm.at[c, pl.ds(s * _RPT, _RPT)], isem).wait()

  return _sc_deg


def _tc1_body(x_ref, deg_ref, w1_ref, z1_ref):
  dinv = lax.rsqrt(deg_ref[0, :, 0:1])
  for c in range(2):
    z1_ref[c] = jnp.dot(x_ref[c], w1_ref[c],
                        preferred_element_type=jnp.float32) * dinv


def _tc2_body(s1_ref, z1_ref, deg_ref, b1_ref, g_ref, be_ref, w2_ref, z2_ref):
  dinv = lax.rsqrt(deg_ref[0, :, 0:1])
  for c in range(2):
    t = dinv * (s1_ref[c] + z1_ref[c]) + b1_ref[c]
    mu = jnp.mean(t, axis=-1, keepdims=True)
    var = jnp.mean((t - mu) ** 2, axis=-1, keepdims=True)
    h = jnp.tanh((t - mu) / jnp.sqrt(var + 1e-5) * g_ref[c] + be_ref[c])
    z2_ref[c] = jnp.dot(h, w2_ref[c],
                        preferred_element_type=jnp.float32) * dinv


def _tc3_body(s2_ref, z2_ref, deg_ref, b2_ref, clf_ref, dw_ref, cw1_ref,
              comb_ref, dom_ref, z3_ref):
  dinv = lax.rsqrt(deg_ref[0, :, 0:1])
  feats = [dinv * (s2_ref[c] + z2_ref[c]) + b2_ref[c] for c in range(2)]
  comb = jnp.concatenate(feats, axis=1)
  comb_ref[...] = comb
  clf = clf_ref[...]
  wn = clf / jnp.maximum(
      jnp.sqrt(jnp.sum(clf * clf, axis=0, keepdims=True)), 1e-12)
  cc = jnp.dot(comb, cw1_ref[...], preferred_element_type=jnp.float32)
  for c in range(2):
    f = feats[c]
    fn = f / jnp.maximum(
        jnp.sqrt(jnp.sum(f * f, axis=1, keepdims=True)), 1e-12)
    dom_ref[c] = 5.0 * jnp.dot(fn, wn, preferred_element_type=jnp.float32)
    dec = jnp.dot(f, dw_ref[c], preferred_element_type=jnp.float32)
    z3_ref[c] = jnp.concatenate(
        [dec, cc[:, c * 32:(c + 1) * 32]], axis=1) * dinv


def _tc4_body(s3_ref, z3_ref, deg_ref, db_ref, cb1_ref, cg_ref, cbe_ref,
              cw2_ref, rec_ref, z4_ref):
  dinv = lax.rsqrt(deg_ref[0, :, 0:1])
  for c in range(2):
    rec_ref[c] = dinv * (s3_ref[c][:, :128] + z3_ref[c][:, :128]) + db_ref[c]
  t = jnp.concatenate([s3_ref[0][:, 128:] + z3_ref[0][:, 128:],
                       s3_ref[1][:, 128:] + z3_ref[1][:, 128:]], axis=1)
  t = dinv * t + cb1_ref[0]
  mask = (lax.broadcasted_iota(jnp.int32, t.shape, 1) < 42).astype(jnp.float32)
  mu = jnp.sum(t, axis=-1, keepdims=True) / 42.0
  d = (t - mu) * mask
  var = jnp.sum(d * d, axis=-1, keepdims=True) / 42.0
  h = jnp.maximum(d / jnp.sqrt(var + 1e-5) * cg_ref[0] + cbe_ref[0], 0.0)
  for c in range(2):
    z4_ref[c] = jnp.dot(h, cw2_ref[c],
                        preferred_element_type=jnp.float32) * dinv


def _tc5_body(s4_ref, z4_ref, deg_ref, cb2_ref, out_ref):
  dinv = lax.rsqrt(deg_ref[0, :, 0:1])
  out_ref[...] = dinv * jnp.concatenate(
      [s4_ref[0] + z4_ref[0], s4_ref[1] + z4_ref[1]], axis=1) + cb2_ref[0]


def _rows(shape):
  return pl.BlockSpec((shape[0], _R) + shape[2:], lambda i: (0, i) + (0,) * (len(shape) - 2))


def _rows2(shape):
  return pl.BlockSpec((_R,) + shape[1:], lambda i: (i,) + (0,) * (len(shape) - 1))


def _full(shape):
  return pl.BlockSpec(shape, lambda i: (0,) * len(shape))


def _tc_call(body, in_arrays, out_shapes, row_split):
  arrs = list(in_arrays)
  specs = []
  for a, split in zip(arrs + list(out_shapes), row_split):
    shape = a.shape
    if split:
      specs.append(_rows(shape) if len(shape) == 3 else _rows2(shape))
    else:
      specs.append(_full(shape))
  n_in = len(arrs)
  return pl.pallas_call(
      body,
      grid=(_NB,),
      in_specs=specs[:n_in],
      out_specs=specs[n_in] if len(out_shapes) == 1 else specs[n_in:],
      out_shape=(out_shapes[0] if len(out_shapes) == 1
                 else tuple(out_shapes)),
  )(*arrs)


def kernel(x_list, batch_list, edge_index, enc0_W1, enc0_b1, enc0_g, enc0_be,
           enc0_W2, enc0_b2, dec0_W, dec0_b, enc1_W1, enc1_b1, enc1_g,
           enc1_be, enc1_W2, enc1_b2, dec1_W, dec1_b, c_W1, c_b1, c_g, c_be,
           c_W2, c_b2, clf_W):
  f32 = jnp.float32
  src = edge_index[0]
  dst = edge_index[1]
  src2 = jnp.stack([src, src + _N]).reshape(_NC, _NCROWS, _CH)
  dstr = dst.reshape(_NCROWS, _CH)
  ones16 = jnp.ones((_RPT, 16), f32)

  w1s = jnp.stack([enc0_W1, enc1_W1])
  b1s = jnp.stack([enc0_b1, enc1_b1])
  gs = jnp.stack([enc0_g, enc1_g])
  bes = jnp.stack([enc0_be, enc1_be])
  w2s = jnp.stack([enc0_W2, enc1_W2])
  b2s = jnp.stack([enc0_b2, enc1_b2])
  dws = jnp.stack([dec0_W, dec1_W])
  dbs = jnp.stack([dec0_b, dec1_b])
  cw1p = jnp.pad(c_W1, ((0, 0), (0, 22)))
  cb1p = jnp.pad(c_b1, (0, 22)).reshape(1, 64)
  cgp = jnp.pad(c_g, (0, 22)).reshape(1, 64)
  cbep = jnp.pad(c_be, (0, 22)).reshape(1, 64)
  cw2p = jnp.pad(c_W2, ((0, 22), (0, 0)))
  cw2s = jnp.stack([cw2p[:, :32], cw2p[:, 32:]])
  cb2r = c_b2.reshape(1, 64)

  deg = _make_sc_deg()(dstr, ones16)

  sds = lambda *s: jax.ShapeDtypeStruct(s, f32)

  z1 = _tc_call(_tc1_body, [x_list, deg, w1s], [sds(2, _N, 128)],
                [1, 1, 0, 1])
  s1 = _make_sc_prop(128)(src2, dstr, z1.reshape(2 * _N, 128),
                          jnp.zeros((_RPT, 128), f32))
  z2 = _tc_call(_tc2_body, [s1, z1, deg, b1s, gs, bes, w2s],
                [sds(2, _N, 64)], [1, 1, 1, 0, 0, 0, 0, 1])
  s2 = _make_sc_prop(64)(src2, dstr, z2.reshape(2 * _N, 64),
                         jnp.zeros((_RPT, 64), f32))
  comb, doms, z3 = _tc_call(
      _tc3_body, [s2, z2, deg, b2s, clf_W, dws, cw1p],
      [sds(_N, 128), sds(2, _N, 8), sds(2, _N, 160)],
      [1, 1, 1, 0, 0, 0, 0, 1, 1, 1])
  s3 = _make_sc_prop(160)(src2, dstr, z3.reshape(2 * _N, 160),
                          jnp.zeros((_RPT, 160), f32))
  recs, z4 = _tc_call(
      _tc4_body, [s3, z3, deg, dbs, cb1p, cgp, cbep, cw2s],
      [sds(2, _N, 128), sds(2, _N, 32)],
      [1, 1, 1, 0, 0, 0, 0, 0, 1, 1])
  s4 = _make_sc_prop(32)(src2, dstr, z4.reshape(2 * _N, 32),
                         jnp.zeros((_RPT, 32), f32))
  comb_out = _tc_call(_tc5_body, [s4, z4, deg, cb2r], [sds(_N, 64)],
                      [1, 1, 1, 0, 1])

  return (comb[:, :64], comb[:, 64:], doms[0], doms[1],
          recs[0], recs[1], comb_out)


# speedup vs baseline: 17.2917x; 1.1251x over previous
import functools

import jax
import jax.numpy as jnp
from jax import lax
from jax.experimental import pallas as pl
from jax.experimental.pallas import tpu as pltpu
from jax.experimental.pallas import tpu_sc as plsc

_N = 10000
_E = 160000
_NC = 2
_NS = 16
_CH = 40
_NCROWS = _E // _CH
_CPT = _NCROWS // _NS
_BLK = 50
_NBLK = _CPT // _BLK
_RPT = _N // _NS

_R = 1000
_NB = _N // _R

_mesh = lambda: plsc.VectorSubcoreMesh(core_axis_name="c", subcore_axis_name="s")
_sc_params = lambda: pltpu.CompilerParams(use_tc_tiling_on_sc=False)


@functools.lru_cache(maxsize=None)
def _make_sc_prop(fh):
  if fh > 128:
    nbuf, blkc = 3, _BLK
  else:
    nbuf, blkc = 6, _CPT

  nblk = _CPT // blkc

  @functools.partial(
      pl.kernel,
      out_type=jax.ShapeDtypeStruct((_NC, _N, fh), jnp.float32),
      mesh=_mesh(),
      scratch_types=[
          pltpu.VMEM((blkc, _CH), jnp.int32),
          pltpu.VMEM((blkc, _CH), jnp.int32),
          pltpu.VMEM((nbuf, _CH, fh), jnp.float32),
          pltpu.VMEM_SHARED((_N, fh), jnp.float32),
          pltpu.SemaphoreType.DMA,
      ] + [pltpu.SemaphoreType.DMA] * (2 * nbuf),
      compiler_params=_sc_params(),
  )
  def prop(src2_hbm, dst_hbm, z_hbm, zeros_hbm, out_hbm,
           srci_v, dsti_v, rows_v, acc_sp, isem, *sems):
    gsem = sems[:nbuf]
    ssem = sems[nbuf:]
    c = lax.axis_index("c")
    s = lax.axis_index("s")

    pltpu.async_copy(zeros_hbm, acc_sp.at[pl.ds(s * _RPT, _RPT)], isem).wait()
    plsc.subcore_barrier()

    def gath(krow, b):
      pltpu.async_copy(z_hbm.at[srci_v.at[krow]], rows_v.at[b], gsem[b])

    def wait_gath(b):
      pltpu.make_async_copy(
          z_hbm.at[srci_v.at[0]], rows_v.at[b], gsem[b]).wait()

    def scat(jrow, b):
      pltpu.async_copy(rows_v.at[b], acc_sp.at[dsti_v.at[jrow]], ssem[b],
                       add=True)

    def wait_scat(b):
      pltpu.make_async_copy(
          rows_v.at[b], acc_sp.at[dsti_v.at[0]], ssem[b]).wait()

    def step(kg, ks, bg, bs):
      wait_scat(bg)
      gath(kg, bg)
      wait_gath(bs)
      scat(ks, bs)

    for blk in range(nblk):
      r0 = s * _CPT + blk * blkc
      pltpu.sync_copy(src2_hbm.at[c, pl.ds(r0, blkc)], srci_v)
      pltpu.sync_copy(dst_hbm.at[pl.ds(r0, blkc)], dsti_v)

      for k in range(nbuf):
        gath(k, k)
      wait_gath(0)
      scat(0, 0)

      ngrp = (blkc - nbuf) // nbuf

      @pl.loop(0, ngrp)
      def _grp(g):
        for i in range(nbuf):
          k = nbuf + g * nbuf + i
          step(k, k - nbuf + 1, i, (i + 1) % nbuf)

      for k in range(nbuf + ngrp * nbuf, blkc):
        step(k, k - nbuf + 1, k % nbuf, (k + 1) % nbuf)

      for j in range(blkc - nbuf + 1, blkc):
        b = j % nbuf
        wait_gath(b)
        scat(j, b)
      for b in range(nbuf):
        wait_scat(b)

    plsc.subcore_barrier()
    pltpu.async_copy(acc_sp.at[pl.ds(s * _RPT, _RPT)],
                     out_hbm.at[c, pl.ds(s * _RPT, _RPT)], isem).wait()

  return prop


@functools.lru_cache(maxsize=None)
def _make_sc_deg():
  @functools.partial(
      pl.kernel,
      out_type=jax.ShapeDtypeStruct((_NC, _N, 16), jnp.float32),
      mesh=_mesh(),
      scratch_types=[
          pltpu.VMEM((_BLK, _CH), jnp.int32),
          pltpu.VMEM((_CH, 16), jnp.float32),
          pltpu.VMEM_SHARED((_N, 16), jnp.float32),
          pltpu.SemaphoreType.DMA,
          pltpu.SemaphoreType.DMA,
      ],
      compiler_params=_sc_params(),
  )
  def _sc_deg(dst_hbm, ones_hbm, out_hbm, didx_v, pay_v, acc_sp, isem, ssem):
    c = lax.axis_index("c")
    s = lax.axis_index("s")

    pltpu.async_copy(ones_hbm, acc_sp.at[pl.ds(s * _RPT, _RPT)], isem).wait()
    pltpu.sync_copy(ones_hbm.at[pl.ds(0, _CH)], pay_v)
    plsc.subcore_barrier()

    for blk in range(_NBLK):
      r0 = s * _CPT + blk * _BLK
      pltpu.sync_copy(dst_hbm.at[pl.ds(r0, _BLK)], didx_v)

      @pl.loop(0, _BLK // 10)
      def _grp(g):
        for i in range(10):
          pltpu.async_copy(pay_v, acc_sp.at[didx_v.at[g * 10 + i]], ssem,
                           add=True)
        for i in range(10):
          pltpu.make_async_copy(pay_v, acc_sp.at[didx_v.at[0]], ssem).wait()

    plsc.subcore_barrier()
    pltpu.async_copy(acc_sp.at[pl.ds(s * _RPT, _RPT)],
                     out_hbm.at[c, pl.ds(s * _RPT, _RPT)], isem).wait()

  return _sc_deg


def _tc1_body(x_ref, deg_ref, w1_ref, z1_ref):
  dinv = lax.rsqrt(deg_ref[0, :, 0:1])
  for c in range(2):
    z1_ref[c] = jnp.dot(x_ref[c], w1_ref[c],
                        preferred_element_type=jnp.float32) * dinv


def _tc2_body(s1_ref, z1_ref, deg_ref, b1_ref, g_ref, be_ref, w2_ref, z2_ref):
  dinv = lax.rsqrt(deg_ref[0, :, 0:1])
  for c in range(2):
    t = dinv * (s1_ref[c] + z1_ref[c]) + b1_ref[c]
    mu = jnp.mean(t, axis=-1, keepdims=True)
    var = jnp.mean((t - mu) ** 2, axis=-1, keepdims=True)
    h = jnp.tanh((t - mu) / jnp.sqrt(var + 1e-5) * g_ref[c] + be_ref[c])
    z2_ref[c] = jnp.dot(h, w2_ref[c],
                        preferred_element_type=jnp.float32) * dinv


def _tc3_body(s2_ref, z2_ref, deg_ref, b2_ref, clf_ref, dw_ref, cw1_ref,
              comb_ref, dom_ref, z3_ref):
  dinv = lax.rsqrt(deg_ref[0, :, 0:1])
  feats = [dinv * (s2_ref[c] + z2_ref[c]) + b2_ref[c] for c in range(2)]
  comb = jnp.concatenate(feats, axis=1)
  comb_ref[...] = comb
  clf = clf_ref[...]
  wn = clf / jnp.maximum(
      jnp.sqrt(jnp.sum(clf * clf, axis=0, keepdims=True)), 1e-12)
  cc = jnp.dot(comb, cw1_ref[...], preferred_element_type=jnp.float32)
  for c in range(2):
    f = feats[c]
    fn = f / jnp.maximum(
        jnp.sqrt(jnp.sum(f * f, axis=1, keepdims=True)), 1e-12)
    dom_ref[c] = 5.0 * jnp.dot(fn, wn, preferred_element_type=jnp.float32)
    dec = jnp.dot(f, dw_ref[c], preferred_element_type=jnp.float32)
    z3_ref[c] = jnp.concatenate(
        [dec, cc[:, c * 32:(c + 1) * 32]], axis=1) * dinv


def _tc4_body(s3_ref, z3_ref, deg_ref, db_ref, cb1_ref, cg_ref, cbe_ref,
              cw2_ref, rec_ref, z4_ref):
  dinv = lax.rsqrt(deg_ref[0, :, 0:1])
  for c in range(2):
    rec_ref[c] = dinv * (s3_ref[c][:, :128] + z3_ref[c][:, :128]) + db_ref[c]
  t = jnp.concatenate([s3_ref[0][:, 128:] + z3_ref[0][:, 128:],
                       s3_ref[1][:, 128:] + z3_ref[1][:, 128:]], axis=1)
  t = dinv * t + cb1_ref[0]
  mask = (lax.broadcasted_iota(jnp.int32, t.shape, 1) < 42).astype(jnp.float32)
  mu = jnp.sum(t, axis=-1, keepdims=True) / 42.0
  d = (t - mu) * mask
  var = jnp.sum(d * d, axis=-1, keepdims=True) / 42.0
  h = jnp.maximum(d / jnp.sqrt(var + 1e-5) * cg_ref[0] + cbe_ref[0], 0.0)
  for c in range(2):
    z4_ref[c] = jnp.dot(h, cw2_ref[c],
                        preferred_element_type=jnp.float32) * dinv


def _tc5_body(s4_ref, z4_ref, deg_ref, cb2_ref, out_ref):
  dinv = lax.rsqrt(deg_ref[0, :, 0:1])
  out_ref[...] = dinv * jnp.concatenate(
      [s4_ref[0] + z4_ref[0], s4_ref[1] + z4_ref[1]], axis=1) + cb2_ref[0]


def _rows(shape):
  return pl.BlockSpec((shape[0], _R) + shape[2:], lambda i: (0, i) + (0,) * (len(shape) - 2))


def _rows2(shape):
  return pl.BlockSpec((_R,) + shape[1:], lambda i: (i,) + (0,) * (len(shape) - 1))


def _full(shape):
  return pl.BlockSpec(shape, lambda i: (0,) * len(shape))


def _tc_call(body, in_arrays, out_shapes, row_split):
  arrs = list(in_arrays)
  specs = []
  for a, split in zip(arrs + list(out_shapes), row_split):
    shape = a.shape
    if split:
      specs.append(_rows(shape) if len(shape) == 3 else _rows2(shape))
    else:
      specs.append(_full(shape))
  n_in = len(arrs)
  return pl.pallas_call(
      body,
      grid=(_NB,),
      in_specs=specs[:n_in],
      out_specs=specs[n_in] if len(out_shapes) == 1 else specs[n_in:],
      out_shape=(out_shapes[0] if len(out_shapes) == 1
                 else tuple(out_shapes)),
  )(*arrs)


def kernel(x_list, batch_list, edge_index, enc0_W1, enc0_b1, enc0_g, enc0_be,
           enc0_W2, enc0_b2, dec0_W, dec0_b, enc1_W1, enc1_b1, enc1_g,
           enc1_be, enc1_W2, enc1_b2, dec1_W, dec1_b, c_W1, c_b1, c_g, c_be,
           c_W2, c_b2, clf_W):
  f32 = jnp.float32
  src = edge_index[0]
  dst = edge_index[1]
  src2 = jnp.stack([src, src + _N]).reshape(_NC, _NCROWS, _CH)
  dstr = dst.reshape(_NCROWS, _CH)
  ones16 = jnp.ones((_RPT, 16), f32)

  w1s = jnp.stack([enc0_W1, enc1_W1])
  b1s = jnp.stack([enc0_b1, enc1_b1])
  gs = jnp.stack([enc0_g, enc1_g])
  bes = jnp.stack([enc0_be, enc1_be])
  w2s = jnp.stack([enc0_W2, enc1_W2])
  b2s = jnp.stack([enc0_b2, enc1_b2])
  dws = jnp.stack([dec0_W, dec1_W])
  dbs = jnp.stack([dec0_b, dec1_b])
  cw1p = jnp.pad(c_W1, ((0, 0), (0, 22)))
  cb1p = jnp.pad(c_b1, (0, 22)).reshape(1, 64)
  cgp = jnp.pad(c_g, (0, 22)).reshape(1, 64)
  cbep = jnp.pad(c_be, (0, 22)).reshape(1, 64)
  cw2p = jnp.pad(c_W2, ((0, 22), (0, 0)))
  cw2s = jnp.stack([cw2p[:, :32], cw2p[:, 32:]])
  cb2r = c_b2.reshape(1, 64)

  deg = _make_sc_deg()(dstr, ones16)

  sds = lambda *s: jax.ShapeDtypeStruct(s, f32)

  z1 = _tc_call(_tc1_body, [x_list, deg, w1s], [sds(2, _N, 128)],
                [1, 1, 0, 1])
  s1 = _make_sc_prop(128)(src2, dstr, z1.reshape(2 * _N, 128),
                          jnp.zeros((_RPT, 128), f32))
  z2 = _tc_call(_tc2_body, [s1, z1, deg, b1s, gs, bes, w2s],
                [sds(2, _N, 64)], [1, 1, 1, 0, 0, 0, 0, 1])
  s2 = _make_sc_prop(64)(src2, dstr, z2.reshape(2 * _N, 64),
                         jnp.zeros((_RPT, 64), f32))
  comb, doms, z3 = _tc_call(
      _tc3_body, [s2, z2, deg, b2s, clf_W, dws, cw1p],
      [sds(_N, 128), sds(2, _N, 8), sds(2, _N, 160)],
      [1, 1, 1, 0, 0, 0, 0, 1, 1, 1])
  s3 = _make_sc_prop(160)(src2, dstr, z3.reshape(2 * _N, 160),
                          jnp.zeros((_RPT, 160), f32))
  recs, z4 = _tc_call(
      _tc4_body, [s3, z3, deg, dbs, cb1p, cgp, cbep, cw2s],
      [sds(2, _N, 128), sds(2, _N, 32)],
      [1, 1, 1, 0, 0, 0, 0, 0, 1, 1])
  s4 = _make_sc_prop(32)(src2, dstr, z4.reshape(2 * _N, 32),
                         jnp.zeros((_RPT, 32), f32))
  comb_out = _tc_call(_tc5_body, [s4, z4, deg, cb2r], [sds(_N, 64)],
                      [1, 1, 1, 0, 1])

  return (comb[:, :64], comb[:, 64:], doms[0], doms[1],
          recs[0], recs[1], comb_out)
